# Initial kernel scaffold; baseline (speedup 1.0000x reference)
import jax
import jax.numpy as jnp
from jax.experimental import pallas as pl


def kernel(feat, edge_index, W_self, W_neigh, b):
    raise NotImplementedError("write your pallas kernel here")


# speedup vs baseline: 7.7672x; 7.7672x over previous
import jax
import jax.numpy as jnp
from jax import lax
from jax.experimental import pallas as pl
from jax.experimental.pallas import tpu as pltpu
from jax.experimental.pallas import tpu_sc as plsc

N_NODES = 10000
N_EDGES = 320000
D_IN = 128
D_OUT = 128

NC = 2
NS = 16
NW = NC * NS
EPW = N_EDGES // NW
CH = 128
NCH = EPW // CH
TAIL = EPW - NCH * CH
RPT = 624
REM = N_NODES - NS * RPT
ZR = 104


def _sc_aggregate_body(feat_hbm, src_hbm, dst_hbm, parts_hbm, degp_hbm,
                       src_idx, dst_idx, rows, src_t, dst_t, rows_t,
                       deg_v, agg, sem):
    cid = lax.axis_index("c")
    sid = lax.axis_index("s")
    wid = cid * NS + sid

    zvec = jnp.zeros((16,), jnp.float32)

    def _zero_row(i, _):
        for j in range(D_IN // 16):
            rows[i, pl.ds(j * 16, 16)] = zvec
        return 0

    lax.fori_loop(0, ZR, _zero_row, 0)

    def _zero_deg(i, _):
        deg_v[pl.ds(i * 16, 16)] = zvec
        return 0

    lax.fori_loop(0, N_NODES // 16, _zero_deg, 0)

    for k in range(RPT // ZR):
        pltpu.sync_copy(rows.at[pl.ds(0, ZR)],
                        agg.at[pl.ds(sid * RPT + k * ZR, ZR)])

    @pl.when(sid == NS - 1)
    def _zero_rem():
        pltpu.sync_copy(rows.at[pl.ds(0, REM)], agg.at[pl.ds(NS * RPT, REM)])

    plsc.subcore_barrier()

    ones16 = jnp.ones((16,), jnp.float32)

    def _chunk(c, _):
        base = wid * EPW + c * CH
        pltpu.sync_copy(src_hbm.at[pl.ds(base, CH)], src_idx)
        pltpu.sync_copy(dst_hbm.at[pl.ds(base, CH)], dst_idx)
        gat = pltpu.async_copy(feat_hbm.at[src_idx], rows, sem)
        for g in range(CH // 16):
            d16 = dst_idx[pl.ds(g * 16, 16)]
            plsc.addupdate_scatter(deg_v, [d16], ones16)
        gat.wait()
        pltpu.sync_copy(rows, agg.at[dst_idx], add=True)
        return 0

    lax.fori_loop(0, NCH, _chunk, 0)

    tbase = wid * EPW + NCH * CH
    pltpu.sync_copy(src_hbm.at[pl.ds(tbase, TAIL)], src_t)
    pltpu.sync_copy(dst_hbm.at[pl.ds(tbase, TAIL)], dst_t)
    pltpu.async_copy(feat_hbm.at[src_t], rows_t, sem).wait()
    d16 = dst_t[pl.ds(0, TAIL)]
    plsc.addupdate_scatter(deg_v, [d16], ones16)
    pltpu.sync_copy(rows_t, agg.at[dst_t], add=True)

    plsc.subcore_barrier()

    pltpu.sync_copy(agg.at[pl.ds(sid * RPT, RPT)],
                    parts_hbm.at[cid, pl.ds(sid * RPT, RPT)])

    @pl.when(sid == NS - 1)
    def _write_rem():
        pltpu.sync_copy(agg.at[pl.ds(NS * RPT, REM)],
                        parts_hbm.at[cid, pl.ds(NS * RPT, REM)])

    pltpu.sync_copy(deg_v, degp_hbm.at[wid])


def _sc_aggregate(feat, src, dst):
    mesh = plsc.VectorSubcoreMesh(core_axis_name="c", subcore_axis_name="s")
    return pl.kernel(
        _sc_aggregate_body,
        out_type=(jax.ShapeDtypeStruct((NC, N_NODES, D_IN), jnp.float32),
                  jax.ShapeDtypeStruct((NW, N_NODES), jnp.float32)),
        mesh=mesh,
        compiler_params=pltpu.CompilerParams(needs_layout_passes=False),
        scratch_types=[
            pltpu.VMEM((CH,), jnp.int32),
            pltpu.VMEM((CH,), jnp.int32),
            pltpu.VMEM((CH, D_IN), jnp.float32),
            pltpu.VMEM((TAIL,), jnp.int32),
            pltpu.VMEM((TAIL,), jnp.int32),
            pltpu.VMEM((TAIL, D_IN), jnp.float32),
            pltpu.VMEM((N_NODES,), jnp.float32),
            pltpu.VMEM_SHARED((N_NODES, D_IN), jnp.float32),
            pltpu.SemaphoreType.DMA,
        ],
    )(feat, src, dst)


BR = 1000


def _tc_epilogue_body(feat_ref, parts_ref, degp_ref, ws_ref, wn_ref, b_ref,
                      out_ref):
    agg = parts_ref[0] + parts_ref[1]
    deg = jnp.sum(degp_ref[...], axis=1)[:, None]
    h_neigh = agg / jnp.maximum(deg, 1.0)
    acc = jnp.dot(feat_ref[...], ws_ref[...], preferred_element_type=jnp.float32)
    acc = acc + jnp.dot(h_neigh, wn_ref[...], preferred_element_type=jnp.float32)
    out_ref[...] = jnp.maximum(acc + b_ref[...], 0.0)


def _tc_epilogue(feat, parts, deg_parts, W_self, W_neigh, b2d):
    return pl.pallas_call(
        _tc_epilogue_body,
        grid=(N_NODES // BR,),
        in_specs=[
            pl.BlockSpec((BR, D_IN), lambda i: (i, 0)),
            pl.BlockSpec((NC, BR, D_IN), lambda i: (0, i, 0)),
            pl.BlockSpec((BR, NW), lambda i: (i, 0)),
            pl.BlockSpec((D_IN, D_OUT), lambda i: (0, 0)),
            pl.BlockSpec((D_IN, D_OUT), lambda i: (0, 0)),
            pl.BlockSpec((1, D_OUT), lambda i: (0, 0)),
        ],
        out_specs=pl.BlockSpec((BR, D_OUT), lambda i: (i, 0)),
        out_shape=jax.ShapeDtypeStruct((N_NODES, D_OUT), jnp.float32),
    )(feat, parts, deg_parts, W_self, W_neigh, b2d)


@jax.jit
def kernel(feat, edge_index, W_self, W_neigh, b):
    src = edge_index[0].astype(jnp.int32)
    dst = edge_index[1].astype(jnp.int32)
    parts, deg_parts = _sc_aggregate(feat, src, dst)
    return _tc_epilogue(feat, parts, deg_parts.T, W_self, W_neigh,
                        b.reshape(1, D_OUT))
